# baseline (speedup 1.0000x reference)
import functools

import jax
import jax.numpy as jnp
from jax import lax
from jax.experimental import pallas as pl
from jax.experimental.pallas import tpu as pltpu
from jax.experimental.pallas import tpu_sc as plsc

_NUM_CORES = 2
_NUM_SUBCORES = 16
_NUM_WORKERS = _NUM_CORES * _NUM_SUBCORES
def _chunk_schedule(rows):
    if rows % 8 == 0 and rows >= 64:
        eighth = rows // 8
        return (eighth, 3 * eighth, rows - 4 * eighth)
    return (rows,)


def _copy_body(table_hbm, out_hbm, bufs, in_sems, out_sems, *, rows_per_w, chunks):
    wid = lax.axis_index("s") * _NUM_CORES + lax.axis_index("c")
    base = wid * rows_per_w

    offs, o = [], 0
    for c in chunks:
        offs.append(o)
        o += c

    reads = []
    for b, (off, c) in enumerate(zip(offs, chunks)):
        r = pltpu.make_async_copy(
            table_hbm.at[pl.ds(base + off, c), :], bufs[b], in_sems[b]
        )
        r.start()
        reads.append(r)

    writes = []
    for b, (off, c) in enumerate(zip(offs, chunks)):
        reads[b].wait()
        w = pltpu.make_async_copy(
            bufs[b], out_hbm.at[pl.ds(base + off, c), :], out_sems[b]
        )
        w.start()
        writes.append(w)

    for w in writes:
        w.wait()


def kernel(x, table):
    seq_len = x.shape[1]
    d = table.shape[1]
    rows_per_w = seq_len // _NUM_WORKERS
    chunks = _chunk_schedule(rows_per_w)
    mesh = plsc.VectorSubcoreMesh(core_axis_name="c", subcore_axis_name="s")

    k = pl.kernel(
        functools.partial(_copy_body, rows_per_w=rows_per_w, chunks=chunks),
        out_type=jax.ShapeDtypeStruct((seq_len, d), jnp.float32),
        mesh=mesh,
        scratch_types=[
            [pltpu.VMEM((c, d), jnp.float32) for c in chunks],
            [pltpu.SemaphoreType.DMA for _ in chunks],
            [pltpu.SemaphoreType.DMA for _ in chunks],
        ],
    )
    return k(table)
